# baseline (speedup 1.0000x reference)
import functools

import jax
import jax.numpy as jnp
from jax import lax
from jax.experimental import pallas as pl
from jax.experimental.pallas import tpu as pltpu
from jax.experimental.pallas import tpu_sc as plsc

_NUM_LABELS = 1000
_EMBED = 64
_NC = 2
_NS = 16
_NW = _NC * _NS
_CHUNK = 128


def _table_body(wt_ref, b_ref, out_ref):
    out_ref[...] = wt_ref[...] + b_ref[...]


def _build_table(Wt, b2):
    return pl.pallas_call(
        _table_body,
        out_shape=jax.ShapeDtypeStruct((_NUM_LABELS, _EMBED), jnp.float32),
    )(Wt, b2)


def _make_gather(n_idx):
    assert n_idx % (_NW * _CHUNK) == 0
    per_w = n_idx // _NW
    n_chunks = per_w // _CHUNK
    mesh = plsc.VectorSubcoreMesh(
        core_axis_name="c", subcore_axis_name="s",
        num_cores=_NC, num_subcores=_NS,
    )

    @functools.partial(
        pl.kernel,
        out_type=jax.ShapeDtypeStruct((n_idx, _EMBED), jnp.float32),
        mesh=mesh,
        scratch_types=[
            pltpu.VMEM((n_chunks, _CHUNK), jnp.int32),
            pltpu.VMEM((per_w, _EMBED), jnp.float32),
            pltpu.SemaphoreType.DMA,
        ],
        compiler_params=pltpu.CompilerParams(use_tc_tiling_on_sc=False),
    )
    def gather(table_hbm, idx_hbm, out_hbm, idx_v, rows_v, sem):
        wid = lax.axis_index("s") * _NC + lax.axis_index("c")
        base = wid * per_w
        pltpu.sync_copy(idx_hbm.at[wid], idx_v)
        copies = [
            pltpu.async_copy(
                table_hbm.at[idx_v.at[c]],
                rows_v.at[pl.ds(c * _CHUNK, _CHUNK)],
                sem,
            )
            for c in range(n_chunks)
        ]
        for cp in copies:
            cp.wait()
        pltpu.sync_copy(rows_v, out_hbm.at[pl.ds(base, per_w)])

    return gather


def kernel(X, W, b):
    B, S, _ = X.shape
    n_idx = B * S
    table = _build_table(W.T, b.reshape(1, _EMBED))
    idx = X.reshape(_NW, n_idx // (_NW * _CHUNK), _CHUNK).astype(jnp.int32)
    out = _make_gather(n_idx)(table, idx)
    return out.reshape(B, S, _EMBED)


# speedup vs baseline: 1.0411x; 1.0411x over previous
import functools

import jax
import jax.numpy as jnp
from jax import lax
from jax.experimental import pallas as pl
from jax.experimental.pallas import tpu as pltpu
from jax.experimental.pallas import tpu_sc as plsc

_NUM_LABELS = 1000
_EMBED = 64
_NC = 2
_NS = 16
_NW = _NC * _NS
_CHUNK = 128


def _make_gather(B, S):
    n_idx = B * S
    assert n_idx % (_NW * _CHUNK) == 0 and B % _NW == 0
    per_w = n_idx // _NW
    n_chunks = per_w // _CHUNK
    b_per_w = B // _NW
    mesh = plsc.VectorSubcoreMesh(
        core_axis_name="c", subcore_axis_name="s",
        num_cores=_NC, num_subcores=_NS,
    )

    @functools.partial(
        pl.kernel,
        out_type=jax.ShapeDtypeStruct((B, S, _EMBED), jnp.float32),
        mesh=mesh,
        scratch_types=[
            pltpu.VMEM((per_w,), jnp.int32),
            pltpu.VMEM((per_w, _EMBED), jnp.float32),
            pltpu.SemaphoreType.DMA,
        ],
        compiler_params=pltpu.CompilerParams(use_tc_tiling_on_sc=False),
    )
    def gather(table_hbm, idx_hbm, out_hbm, idx_v, rows_v, sem):
        wid = lax.axis_index("s") * _NC + lax.axis_index("c")
        pltpu.sync_copy(idx_hbm.at[pl.ds(wid * per_w, per_w)], idx_v)
        copies = [
            pltpu.async_copy(
                table_hbm.at[idx_v.at[pl.ds(c * _CHUNK, _CHUNK)]],
                rows_v.at[pl.ds(c * _CHUNK, _CHUNK)],
                sem,
            )
            for c in range(n_chunks)
        ]
        for cp in copies:
            cp.wait()
        outs = [
            pltpu.async_copy(
                rows_v.at[pl.ds(p * S, S)],
                out_hbm.at[wid * b_per_w + p],
                sem,
            )
            for p in range(b_per_w)
        ]
        for cp in outs:
            cp.wait()

    return gather


def kernel(X, W, b):
    B, S, _ = X.shape
    table = W.T + b[None, :]
    idx = X.reshape(B * S).astype(jnp.int32)
    return _make_gather(B, S)(table, idx)
